# Initial kernel scaffold; baseline (speedup 1.0000x reference)
import functools

import jax
import jax.numpy as jnp
from jax import lax
from jax.experimental import pallas as pl
from jax.experimental.pallas import tpu as pltpu
from jax.experimental.pallas import tpu_sc as plsc

_D = 64
_NC = 2
_NS = 16
_NW = _NC * _NS
_SUB = 128


@functools.partial(jax.jit, static_argnums=(2, 3))
def _emb_lookup(idx_flat, weight, b_per_w, chunk):
    n_chunks = b_per_w // chunk
    n_sub = chunk // _SUB
    total = b_per_w * _NW

    mesh = plsc.VectorSubcoreMesh(core_axis_name="c", subcore_axis_name="s")

    @functools.partial(
        pl.kernel,
        out_type=jax.ShapeDtypeStruct((total, _D), jnp.float32),
        mesh=mesh,
        scratch_types=[
            pltpu.VMEM((b_per_w,), jnp.int32),
            pltpu.VMEM((chunk, _D), jnp.float32),
            pltpu.SemaphoreType.DMA,
        ],
    )
    def body(idx_hbm, table_hbm, out_hbm, idx_v, rows_v, gsem):
        wid = lax.axis_index("s") * _NC + lax.axis_index("c")
        base = wid * b_per_w
        pltpu.sync_copy(idx_hbm.at[pl.ds(base, b_per_w)], idx_v)

        def chunk_body(g, _):
            off = g * chunk
            copies = []
            for j in range(n_sub):
                copies.append(
                    pltpu.async_copy(
                        table_hbm.at[idx_v.at[pl.ds(off + j * _SUB, _SUB)]],
                        rows_v.at[pl.ds(j * _SUB, _SUB)],
                        gsem,
                    )
                )
            for c in copies:
                c.wait()
            pltpu.sync_copy(rows_v, out_hbm.at[pl.ds(base + off, chunk)])
            return 0

        lax.fori_loop(0, n_chunks, chunk_body, 0)

    return body(idx_flat, weight)


def kernel(input_tokens, weight):
    b, t = input_tokens.shape
    idx_flat = input_tokens.reshape(-1).astype(jnp.int32)
    total = b * t
    b_per_w = total // _NW
    out = _emb_lookup(idx_flat, weight, b_per_w, 640)
    return out.reshape(b, t, _D)


# speedup vs baseline: 1.8425x; 1.8425x over previous
import functools

import jax
import jax.numpy as jnp
from jax import lax
from jax.experimental import pallas as pl
from jax.experimental.pallas import tpu as pltpu
from jax.experimental.pallas import tpu_sc as plsc

_D = 64
_NC = 2
_NS = 16
_NW = _NC * _NS
_SUB = 128


@functools.partial(jax.jit, static_argnums=(2, 3))
def _emb_lookup(idx_flat, weight, b_per_w, chunk):
    n_chunks = b_per_w // chunk
    n_sub = chunk // _SUB
    total = b_per_w * _NW

    mesh = plsc.VectorSubcoreMesh(core_axis_name="c", subcore_axis_name="s")

    @functools.partial(
        pl.kernel,
        out_type=jax.ShapeDtypeStruct((total, _D), jnp.float32),
        mesh=mesh,
        scratch_types=[
            pltpu.VMEM((b_per_w,), jnp.int32),
            pltpu.VMEM((chunk, _D), jnp.float32),
            pltpu.SemaphoreType.DMA,
        ],
        compiler_params=pltpu.CompilerParams(use_tc_tiling_on_sc=False),
    )
    def body(idx_hbm, table_hbm, out_hbm, idx_v, rows_v, gsem):
        wid = lax.axis_index("s") * _NC + lax.axis_index("c")
        base = wid * b_per_w
        pltpu.sync_copy(idx_hbm.at[pl.ds(base, b_per_w)], idx_v)

        def chunk_body(g, _):
            off = g * chunk
            copies = []
            for j in range(n_sub):
                copies.append(
                    pltpu.async_copy(
                        table_hbm.at[idx_v.at[pl.ds(off + j * _SUB, _SUB)]],
                        rows_v.at[pl.ds(j * _SUB, _SUB)],
                        gsem,
                    )
                )
            for c in copies:
                c.wait()
            pltpu.sync_copy(rows_v, out_hbm.at[pl.ds(base + off, chunk)])
            return 0

        lax.fori_loop(0, n_chunks, chunk_body, 0)

    return body(idx_flat, weight)


def kernel(input_tokens, weight):
    b, t = input_tokens.shape
    idx_flat = input_tokens.reshape(-1).astype(jnp.int32)
    total = b * t
    b_per_w = total // _NW
    out = _emb_lookup(idx_flat, weight, b_per_w, 640)
    return out.reshape(b, t, _D)


# speedup vs baseline: 1.8691x; 1.0144x over previous
import functools

import jax
import jax.numpy as jnp
from jax import lax
from jax.experimental import pallas as pl
from jax.experimental.pallas import tpu as pltpu
from jax.experimental.pallas import tpu_sc as plsc

_D = 64
_NC = 2
_NS = 16
_NW = _NC * _NS
_SUB = 128


@functools.partial(jax.jit, static_argnums=(2, 3, 4))
def _emb_lookup(idx_flat, weight, b_per_w, chunk, nbuf):
    n_chunks = b_per_w // chunk
    n_sub = chunk // _SUB
    n_ring = n_chunks // nbuf
    total = b_per_w * _NW

    mesh = plsc.VectorSubcoreMesh(core_axis_name="c", subcore_axis_name="s")

    @functools.partial(
        pl.kernel,
        out_type=jax.ShapeDtypeStruct((total, _D), jnp.float32),
        mesh=mesh,
        scratch_types=[
            pltpu.VMEM((b_per_w,), jnp.int32),
            [pltpu.VMEM((chunk, _D), jnp.float32) for _ in range(nbuf)],
            [pltpu.SemaphoreType.DMA for _ in range(nbuf)],
            [pltpu.SemaphoreType.DMA for _ in range(nbuf)],
        ],
        compiler_params=pltpu.CompilerParams(use_tc_tiling_on_sc=False),
    )
    def body(idx_hbm, table_hbm, out_hbm, idx_v, rows, gsem, ssem):
        wid = lax.axis_index("s") * _NC + lax.axis_index("c")
        base = wid * b_per_w
        pltpu.sync_copy(idx_hbm.at[pl.ds(base, b_per_w)], idx_v)

        def fire_gathers(g, b):
            off = g * chunk
            for j in range(n_sub):
                pltpu.async_copy(
                    table_hbm.at[idx_v.at[pl.ds(off + j * _SUB, _SUB)]],
                    rows[b].at[pl.ds(j * _SUB, _SUB)],
                    gsem[b],
                )

        def wait_gathers(b):
            for j in range(n_sub):
                pltpu.make_async_copy(
                    table_hbm.at[idx_v.at[pl.ds(j * _SUB, _SUB)]],
                    rows[b].at[pl.ds(j * _SUB, _SUB)],
                    gsem[b],
                ).wait()

        for b in range(nbuf):
            fire_gathers(b, b)

        def ring_body(t, _):
            for b in range(nbuf):
                off = (t * nbuf + b) * chunk
                wait_gathers(b)
                pltpu.async_copy(
                    rows[b], out_hbm.at[pl.ds(base + off, chunk)], ssem[b]
                )
            for b in range(nbuf):
                g_next = (t + 1) * nbuf + b
                pltpu.make_async_copy(
                    rows[b], out_hbm.at[pl.ds(base, chunk)], ssem[b]
                ).wait()

                @pl.when(g_next < n_chunks)
                def _():
                    fire_gathers(g_next, b)

            return 0

        lax.fori_loop(0, n_ring, ring_body, 0)

    return body(idx_flat, weight)


def kernel(input_tokens, weight):
    b, t = input_tokens.shape
    idx_flat = input_tokens.reshape(-1).astype(jnp.int32)
    total = b * t
    b_per_w = total // _NW
    out = _emb_lookup(idx_flat, weight, b_per_w, 256, 4)
    return out.reshape(b, t, _D)
